# baseline (speedup 1.0000x reference)
import functools

import jax
import jax.numpy as jnp
from jax import lax
from jax.experimental import pallas as pl
from jax.experimental.pallas import tpu as pltpu
from jax.experimental.pallas import tpu_sc as plsc

_NEG_INF = -1e30


def _lane_allsum(v, lane):
    for sh in (8, 4, 2, 1):
        perm = jax.lax.bitwise_xor(lane, sh)
        v = v + v.at[perm].get(mode="promise_in_bounds")
    return v


def _sc_scores_kernel(B, C, D, V):
    info = plsc.get_sparse_core_info()
    NC, NS = info.num_cores, info.num_subcores
    NW = NC * NS
    CHUNK = B // NW
    S = 32
    NSUB = CHUNK // S
    IDXR = (S * C) // 128
    POSR = CHUNK // 128
    G = D // 16
    inv_c = 1.0 / C

    mesh = plsc.VectorSubcoreMesh(core_axis_name="c", subcore_axis_name="s")

    @functools.partial(
        pl.kernel,
        out_type=jax.ShapeDtypeStruct((B, 16), jnp.float32),
        mesh=mesh,
        scratch_types=[
            pltpu.VMEM((S * C,), jnp.int32),
            pltpu.VMEM((S * C, D), jnp.float32),
            pltpu.VMEM((CHUNK,), jnp.int32),
            pltpu.VMEM((CHUNK, D), jnp.float32),
            pltpu.VMEM((8,), jnp.int32),
            pltpu.VMEM((8, D), jnp.float32),
            pltpu.VMEM((CHUNK, 16), jnp.float32),
            pltpu.SemaphoreType.DMA,
        ],
        compiler_params=pltpu.CompilerParams(use_tc_tiling_on_sc=False),
    )
    def scores_kernel(x_hbm, pos_hbm, neg_hbm, w_hbm, out_hbm,
                      xidx_v, rows_v, pidx_v, pos_v, nidx_v, neg_v, sc_v,
                      sem):
        cid = lax.axis_index("c")
        sid = lax.axis_index("s")
        wid = sid * NC + cid
        base = wid * CHUNK

        lane = lax.broadcasted_iota(jnp.int32, (16,), 0)

        pltpu.sync_copy(neg_hbm, nidx_v)
        pltpu.async_copy(w_hbm.at[nidx_v], neg_v, sem).wait()

        pltpu.sync_copy(pos_hbm.at[pl.ds(base, CHUNK)], pidx_v)
        hs = [
            pltpu.async_copy(w_hbm.at[pidx_v.at[pl.ds(q * 128, 128)]],
                             pos_v.at[pl.ds(q * 128, 128)], sem)
            for q in range(POSR)
        ]
        for h in hs:
            h.wait()

        def sub_chunk(j, _):
            pltpu.sync_copy(x_hbm.at[pl.ds(base * C + j * (S * C), S * C)],
                            xidx_v)
            gs = [
                pltpu.async_copy(w_hbm.at[xidx_v.at[pl.ds(q * 128, 128)]],
                                 rows_v.at[pl.ds(q * 128, 128)], sem)
                for q in range(IDXR)
            ]
            for g in gs:
                g.wait()

            def row_body(i, _):
                r = j * S + i
                rb = i * C
                acc = [rows_v[rb, pl.ds(g * 16, 16)] for g in range(G)]
                for cc in range(1, C):
                    for g in range(G):
                        acc[g] = acc[g] + rows_v[rb + cc, pl.ds(g * 16, 16)]

                ps = acc[0] * pos_v[r, pl.ds(0, 16)]
                for g in range(1, G):
                    ps = ps + acc[g] * pos_v[r, pl.ds(g * 16, 16)]
                s16 = jnp.full((16,), _NEG_INF, jnp.float32)
                s16 = jnp.where(lane == 0, _lane_allsum(ps, lane) * inv_c,
                                s16)

                for k in range(5):
                    ns = acc[0] * neg_v[k, pl.ds(0, 16)]
                    for g in range(1, G):
                        ns = ns + acc[g] * neg_v[k, pl.ds(g * 16, 16)]
                    s16 = jnp.where(lane == k + 1,
                                    _lane_allsum(ns, lane) * inv_c, s16)

                sc_v[r, :] = s16
                return 0

            lax.fori_loop(0, S, row_body, 0)
            return 0

        lax.fori_loop(0, NSUB, sub_chunk, 0)
        pltpu.sync_copy(sc_v, out_hbm.at[pl.ds(base, CHUNK)])

    return scores_kernel


def _tc_loss_kernel(scores_ref, out_ref):
    s = scores_ref[...]
    m = jnp.max(s, axis=1, keepdims=True)
    e = jnp.exp(s - m)
    lse = jnp.log(jnp.sum(e, axis=1)) + m[:, 0]
    out_ref[0, 0] = jnp.mean(lse - s[:, 0])


def kernel(x, pos_labels, neg_labels, W):
    B, C = x.shape
    V, D = W.shape
    x_flat = x.reshape(B * C)
    neg8 = jnp.concatenate(
        [neg_labels, jnp.zeros((8 - neg_labels.shape[0],), jnp.int32)])

    scores = _sc_scores_kernel(B, C, D, V)(x_flat, pos_labels, neg8, W)

    loss = pl.pallas_call(
        _tc_loss_kernel,
        out_shape=jax.ShapeDtypeStruct((1, 1), jnp.float32),
        out_specs=pl.BlockSpec(memory_space=pltpu.SMEM),
    )(scores)
    return loss[0, 0]
